# baseline (speedup 1.0000x reference)
import functools

import jax
import jax.numpy as jnp
from jax import lax
from jax.experimental import pallas as pl
from jax.experimental.pallas import tpu as pltpu
from jax.experimental.pallas import tpu_sc as plsc


@functools.lru_cache(maxsize=None)
def _build_lookup(N: int, V: int, D: int):
    info = plsc.get_sparse_core_info()
    NC, NS, L = info.num_cores, info.num_subcores, info.num_lanes
    NW = NC * NS
    assert D == L and N % NW == 0
    per_w = N // NW

    C = 3200
    while per_w % (2 * C) != 0:
        C //= 2
    iters = per_w // C
    pairs = iters // 2
    G = C // L

    mesh = plsc.VectorSubcoreMesh(core_axis_name="c", subcore_axis_name="s")

    @functools.partial(
        pl.kernel,
        mesh=mesh,
        out_type=jax.ShapeDtypeStruct((N * D,), jnp.float32),
        scratch_types=[
            pltpu.VMEM((V * D,), jnp.float32),
            pltpu.VMEM((C,), jnp.int32),
            pltpu.VMEM((C,), jnp.int32),
            pltpu.VMEM((C * D,), jnp.float32),
            pltpu.VMEM((C * D,), jnp.float32),
            pltpu.SemaphoreType.DMA,
            pltpu.SemaphoreType.DMA,
            pltpu.SemaphoreType.DMA,
            pltpu.SemaphoreType.DMA,
        ],
        compiler_params=pltpu.CompilerParams(
            use_tc_tiling_on_sc=False, needs_layout_passes=False),
    )
    def lookup(seq_hbm, table_hbm, out_hbm, table_v, idx0, idx1, rows0, rows1,
               sem_i0, sem_i1, sem_o0, sem_o1):
        wid = lax.axis_index("s") * NC + lax.axis_index("c")
        base = wid * per_w
        iota = lax.iota(jnp.int32, L)

        pltpu.sync_copy(table_hbm, table_v)
        pltpu.async_copy(seq_hbm.at[pl.ds(base, C)], idx0, sem_i0)
        pltpu.async_copy(seq_hbm.at[pl.ds(base + C, C)], idx1, sem_i1)

        iota16 = iota * L

        def expand(idx_ref, rows_ref):
            def group(j, carry):
                seqv = idx_ref[pl.ds(j * L, L)]
                sc = seqv * L
                obase = iota16 + j * (L * L)
                for d in range(L):
                    perm = (iota + d) & (L - 1)
                    col = plsc.load_gather(table_v, [sc + perm])
                    plsc.store_scatter(rows_ref, [obase + perm], col)
                return carry

            lax.fori_loop(0, G, group, 0, unroll=4)

        def half(g, idx_ref, rows_ref, sem_i, sem_o):
            pltpu.make_async_copy(
                seq_hbm.at[pl.ds(base, C)], idx_ref, sem_i).wait()

            @pl.when(g >= 2)
            def _():
                pltpu.make_async_copy(
                    rows_ref, out_hbm.at[pl.ds(base * D, C * D)], sem_o).wait()

            expand(idx_ref, rows_ref)
            pltpu.async_copy(
                rows_ref, out_hbm.at[pl.ds((base + g * C) * D, C * D)], sem_o)

            @pl.when(g + 2 < iters)
            def _():
                pltpu.async_copy(
                    seq_hbm.at[pl.ds(base + (g + 2) * C, C)], idx_ref, sem_i)

        def pair(p, carry):
            half(2 * p, idx0, rows0, sem_i0, sem_o0)
            half(2 * p + 1, idx1, rows1, sem_i1, sem_o1)
            return carry

        lax.fori_loop(0, pairs, pair, 0)
        pltpu.make_async_copy(
            rows0, out_hbm.at[pl.ds(base * D, C * D)], sem_o0).wait()
        pltpu.make_async_copy(
            rows1, out_hbm.at[pl.ds(base * D, C * D)], sem_o1).wait()

    return lookup


def kernel(seq, emb_weight):
    B, T = seq.shape
    V, D = emb_weight.shape
    N = B * T
    flat = seq.reshape(N).astype(jnp.int32)
    out = _build_lookup(N, V, D)(flat, emb_weight.reshape(V * D))
    return out.reshape(B, T, D)


# speedup vs baseline: 3.4747x; 3.4747x over previous
import functools

import jax
import jax.numpy as jnp
from jax import lax
from jax.experimental import pallas as pl
from jax.experimental.pallas import tpu as pltpu
from jax.experimental.pallas import tpu_sc as plsc


@functools.lru_cache(maxsize=None)
def _build_lookup(B: int, T: int, V: int, D: int):
    info = plsc.get_sparse_core_info()
    NC, NS, L = info.num_cores, info.num_subcores, info.num_lanes
    NW = NC * NS
    SUB, LANE = 8, 128
    assert D == 2 * SUB and B % (NW * LANE) == 0 and T % SUB == 0
    TT = T // SUB
    BPW = B // (NW * LANE)
    UNITS = TT * BPW
    assert UNITS % 2 == 0
    K = LANE // L

    mesh = plsc.VectorSubcoreMesh(core_axis_name="c", subcore_axis_name="s")

    @functools.partial(
        pl.kernel,
        mesh=mesh,
        out_type=jax.ShapeDtypeStruct((T, D, B), jnp.float32),
        scratch_types=[
            pltpu.VMEM((V * D,), jnp.float32),
            pltpu.VMEM((SUB, LANE), jnp.int32),
            pltpu.VMEM((SUB, LANE), jnp.int32),
            pltpu.VMEM((SUB, D, LANE), jnp.float32),
            pltpu.VMEM((SUB, D, LANE), jnp.float32),
            pltpu.SemaphoreType.DMA,
            pltpu.SemaphoreType.DMA,
            pltpu.SemaphoreType.DMA,
            pltpu.SemaphoreType.DMA,
        ],
        compiler_params=pltpu.CompilerParams(needs_layout_passes=False),
    )
    def lookup(seq_hbm, tab_hbm, out_hbm, tab_v, seq0, seq1, outb0, outb1,
               sem_i0, sem_i1, sem_o0, sem_o1):
        wid = lax.axis_index("s") * NC + lax.axis_index("c")
        bbase = wid * (BPW * LANE)

        pltpu.sync_copy(tab_hbm, tab_v)

        def unit_pos(u):
            t0 = (u // BPW) * SUB
            b0 = bbase + (u % BPW) * LANE
            return t0, b0

        def start_in(u, seq_b, sem_i):
            t0, b0 = unit_pos(u)
            pltpu.async_copy(
                seq_hbm.at[pl.ds(t0, SUB), pl.ds(b0, LANE)], seq_b, sem_i)

        start_in(0, seq0, sem_i0)
        start_in(1, seq1, sem_i1)

        def expand(seq_b, out_b):
            for t in range(SUB):
                for k in range(K):
                    sv = seq_b[t, pl.ds(k * L, L)]
                    for d in range(D):
                        col = plsc.load_gather(tab_v, [sv + d * V])
                        out_b[t, d, pl.ds(k * L, L)] = col

        def half(u, seq_b, out_b, sem_i, sem_o):
            pltpu.make_async_copy(
                seq_hbm.at[pl.ds(0, SUB), pl.ds(0, LANE)], seq_b, sem_i).wait()

            @pl.when(u >= 2)
            def _():
                pltpu.make_async_copy(
                    out_b, out_hbm.at[pl.ds(0, SUB), pl.ds(0, D),
                                      pl.ds(0, LANE)], sem_o).wait()

            expand(seq_b, out_b)
            t0, b0 = unit_pos(u)
            pltpu.async_copy(
                out_b,
                out_hbm.at[pl.ds(t0, SUB), pl.ds(0, D), pl.ds(b0, LANE)],
                sem_o)

            @pl.when(u + 2 < UNITS)
            def _():
                start_in(u + 2, seq_b, sem_i)

        def pair(p, carry):
            half(2 * p, seq0, outb0, sem_i0, sem_o0)
            half(2 * p + 1, seq1, outb1, sem_i1, sem_o1)
            return carry

        lax.fori_loop(0, UNITS // 2, pair, 0)
        pltpu.make_async_copy(
            outb0, out_hbm.at[pl.ds(0, SUB), pl.ds(0, D), pl.ds(0, LANE)],
            sem_o0).wait()
        pltpu.make_async_copy(
            outb1, out_hbm.at[pl.ds(0, SUB), pl.ds(0, D), pl.ds(0, LANE)],
            sem_o1).wait()

    return lookup


def kernel(seq, emb_weight):
    B, T = seq.shape
    V, D = emb_weight.shape
    seq_t = jnp.transpose(seq).astype(jnp.int32)
    tab_cm = jnp.transpose(emb_weight).reshape(V * D)
    out_t = _build_lookup(B, T, V, D)(seq_t, tab_cm)
    return jnp.transpose(out_t, (2, 0, 1))


# speedup vs baseline: 6.1536x; 1.7710x over previous
import functools

import jax
import jax.numpy as jnp
from jax import lax
from jax.experimental import pallas as pl
from jax.experimental.pallas import tpu as pltpu
from jax.experimental.pallas import tpu_sc as plsc


@functools.lru_cache(maxsize=None)
def _build_lookup(B: int, T: int, V: int, D: int):
    info = plsc.get_sparse_core_info()
    NC, NS, L = info.num_cores, info.num_subcores, info.num_lanes
    NW = NC * NS
    SUB, LANE = 8, 128
    assert D == 2 * SUB and B % (NW * LANE) == 0 and T % SUB == 0
    TT = T // SUB
    BPW = B // (NW * LANE)
    UNITS = TT * BPW
    assert UNITS % 2 == 0
    K = LANE // L

    mesh = plsc.VectorSubcoreMesh(core_axis_name="c", subcore_axis_name="s")

    @functools.partial(
        pl.kernel,
        mesh=mesh,
        out_type=jax.ShapeDtypeStruct((T, D, B), jnp.float32),
        scratch_types=[
            pltpu.VMEM((V * D,), jnp.float32),
            pltpu.VMEM((SUB, LANE), jnp.int32),
            pltpu.VMEM((SUB, LANE), jnp.int32),
            pltpu.VMEM((SUB, D, LANE), jnp.float32),
            pltpu.VMEM((SUB, D, LANE), jnp.float32),
            pltpu.SemaphoreType.DMA,
            pltpu.SemaphoreType.DMA,
            pltpu.SemaphoreType.DMA,
            pltpu.SemaphoreType.DMA,
        ],
        compiler_params=pltpu.CompilerParams(needs_layout_passes=False),
    )
    def lookup(seq_hbm, tab_hbm, out_hbm, tab_v, seq0, seq1, outb0, outb1,
               sem_i0, sem_i1, sem_o0, sem_o1):
        wid = lax.axis_index("s") * NC + lax.axis_index("c")
        bbase = wid * (BPW * LANE)

        pltpu.sync_copy(tab_hbm, tab_v)

        def unit_pos(u):
            t0 = (u // BPW) * SUB
            b0 = bbase + (u % BPW) * LANE
            return t0, b0

        def start_in(u, seq_b, sem_i):
            t0, b0 = unit_pos(u)
            pltpu.async_copy(
                seq_hbm.at[pl.ds(t0, SUB), pl.ds(b0, LANE)], seq_b, sem_i)

        start_in(0, seq0, sem_i0)
        start_in(1, seq1, sem_i1)

        def expand(seq_b, out_b):
            for t in range(SUB):
                for k in range(K):
                    sv = seq_b[t, pl.ds(k * L, L)]
                    cols = [plsc.load_gather(tab_v, [sv + d * V])
                            for d in range(D)]
                    for d in range(D):
                        out_b[t, d, pl.ds(k * L, L)] = cols[d]

        def half(u, seq_b, out_b, sem_i, sem_o):
            pltpu.make_async_copy(
                seq_hbm.at[pl.ds(0, SUB), pl.ds(0, LANE)], seq_b, sem_i).wait()

            @pl.when(u >= 2)
            def _():
                pltpu.make_async_copy(
                    out_b, out_hbm.at[pl.ds(0, SUB), pl.ds(0, D),
                                      pl.ds(0, LANE)], sem_o).wait()

            expand(seq_b, out_b)
            t0, b0 = unit_pos(u)
            pltpu.async_copy(
                out_b,
                out_hbm.at[pl.ds(t0, SUB), pl.ds(0, D), pl.ds(b0, LANE)],
                sem_o)

            @pl.when(u + 2 < UNITS)
            def _():
                start_in(u + 2, seq_b, sem_i)

        def pair(p, carry):
            half(2 * p, seq0, outb0, sem_i0, sem_o0)
            half(2 * p + 1, seq1, outb1, sem_i1, sem_o1)
            return carry

        lax.fori_loop(0, UNITS // 2, pair, 0)
        pltpu.make_async_copy(
            outb0, out_hbm.at[pl.ds(0, SUB), pl.ds(0, D), pl.ds(0, LANE)],
            sem_o0).wait()
        pltpu.make_async_copy(
            outb1, out_hbm.at[pl.ds(0, SUB), pl.ds(0, D), pl.ds(0, LANE)],
            sem_o1).wait()

    return lookup


def kernel(seq, emb_weight):
    B, T = seq.shape
    V, D = emb_weight.shape
    seq_t = jnp.transpose(seq).astype(jnp.int32)
    tab_cm = jnp.transpose(emb_weight).reshape(V * D)
    out_t = _build_lookup(B, T, V, D)(seq_t, tab_cm)
    return jnp.transpose(out_t, (2, 0, 1))
